# Initial kernel scaffold; baseline (speedup 1.0000x reference)
import jax
import jax.numpy as jnp
from jax.experimental import pallas as pl


def kernel(features, edge_index, W_self1, W_neigh1, b1, W_self2, W_neigh2, b2, W_fc, b_fc):
    raise NotImplementedError("write your pallas kernel here")


# speedup vs baseline: 5.6613x; 5.6613x over previous
import jax
import jax.numpy as jnp
from jax import lax
from jax.experimental import pallas as pl
from jax.experimental.pallas import tpu as pltpu
from jax.experimental.pallas import tpu_sc as plsc

_N = 10000
_E = 320000
_D = 128
_DH = _D // 2

_NC = 2
_NS = 16
_EPW = _E // _NS
_GS = 80
_NG = _EPW // _GS
_NP = 10240
_RPS = _NP // _NS

_mesh = plsc.VectorSubcoreMesh(core_axis_name="c", subcore_axis_name="s")


def _make_sc_pass(with_deg):
    scratch = [
        pltpu.VMEM((_NG, _GS), jnp.int32),
        pltpu.VMEM((_NG, _GS), jnp.int32),
        pltpu.VMEM((_GS, _DH), jnp.float32),
        pltpu.VMEM((_NP,), jnp.float32),
        pltpu.VMEM_SHARED((_NP, _DH), jnp.float32),
        pltpu.SemaphoreType.DMA,
    ]
    if with_deg:
        out_type = (
            jax.ShapeDtypeStruct((_NC, _NP, _DH), jnp.float32),
            jax.ShapeDtypeStruct((_NS, _NP), jnp.float32),
        )
    else:
        out_type = jax.ShapeDtypeStruct((_NC, _NP, _DH), jnp.float32)

    def body(x2_hbm, src_hbm, dst_hbm, z_agg_hbm, *rest):
        if with_deg:
            (agg_out, deg_out, src_v, dst_v, rows_v, deg_v,
             agg_sh, sem) = rest
        else:
            (agg_out, src_v, dst_v, rows_v, deg_v,
             agg_sh, sem) = rest
            deg_out = None
        c = lax.axis_index("c")
        s = lax.axis_index("s")
        rows = pl.ds(s * _RPS, _RPS)

        pltpu.sync_copy(src_hbm.at[s], src_v)
        pltpu.sync_copy(dst_hbm.at[s], dst_v)
        pltpu.sync_copy(z_agg_hbm.at[rows], agg_sh.at[rows])
        if with_deg:
            zeros16 = jnp.zeros((16,), jnp.float32)

            def zstep(i, carry):
                deg_v[pl.ds(i * 16, 16)] = zeros16
                return carry

            lax.fori_loop(0, _NP // 16, zstep, 0)
        plsc.subcore_barrier()

        ones16 = jnp.ones((16,), jnp.float32)

        def step(g, carry):
            pltpu.async_copy(x2_hbm.at[c].at[src_v.at[g]], rows_v, sem).wait()
            pltpu.sync_copy(rows_v, agg_sh.at[dst_v.at[g]], add=True)
            if with_deg:
                @pl.when(c == 0)
                def _():
                    for j in range(_GS // 16):
                        idx = dst_v[g, pl.ds(j * 16, 16)]
                        plsc.addupdate_scatter(deg_v, [idx], ones16)
            return carry

        lax.fori_loop(0, _NG, step, 0)
        plsc.subcore_barrier()

        pltpu.sync_copy(agg_sh.at[rows], agg_out.at[c, rows])
        if with_deg:
            @pl.when(c == 0)
            def _():
                pltpu.sync_copy(deg_v, deg_out.at[s])

    return pl.kernel(body, out_type=out_type, mesh=_mesh,
                     scratch_types=scratch,
                     compiler_params=pltpu.CompilerParams(
                         needs_layout_passes=False,
                         use_tc_tiling_on_sc=False))


_sc_pass_deg = _make_sc_pass(True)
_sc_pass = _make_sc_pass(False)

_BR = 1024


def _combine(agg_ref, deg_ref):
    agg = jnp.concatenate([agg_ref[0], agg_ref[1]], axis=1)
    deg = jnp.sum(deg_ref[...], axis=0)[:, None]
    return agg / jnp.maximum(deg, 1.0)


def _tc_layer1_body(x_ref, agg_ref, deg_ref, ws_ref, wn_ref, b_ref, out_ref):
    nbar = _combine(agg_ref, deg_ref)
    out_ref[...] = (
        jnp.dot(x_ref[...], ws_ref[...], preferred_element_type=jnp.float32)
        + jnp.dot(nbar, wn_ref[...], preferred_element_type=jnp.float32)
        + b_ref[...]
    )


def _tc_layer2_body(h_ref, agg_ref, deg_ref, ws_ref, wn_ref, b_ref,
                    wfc_ref, bfc_ref, out_ref):
    nbar = _combine(agg_ref, deg_ref)
    h2 = (
        jnp.dot(h_ref[...], ws_ref[...], preferred_element_type=jnp.float32)
        + jnp.dot(nbar, wn_ref[...], preferred_element_type=jnp.float32)
        + b_ref[...]
    )
    out_ref[...] = (
        jnp.dot(h2, wfc_ref[...], preferred_element_type=jnp.float32)
        + bfc_ref[...]
    )


def _row_spec():
    return pl.BlockSpec((_BR, _D), lambda i: (i, 0))


def _agg_spec():
    return pl.BlockSpec((_NC, _BR, _DH), lambda i: (0, i, 0))


def _deg_spec():
    return pl.BlockSpec((_NS, _BR), lambda i: (0, i))


def _w_spec():
    return pl.BlockSpec((_D, _D), lambda i: (0, 0))


def _b_spec():
    return pl.BlockSpec((1, _D), lambda i: (0, 0))


_tc_layer1 = pl.pallas_call(
    _tc_layer1_body,
    grid=(_NP // _BR,),
    in_specs=[_row_spec(), _agg_spec(), _deg_spec(), _w_spec(), _w_spec(),
              _b_spec()],
    out_specs=_row_spec(),
    out_shape=jax.ShapeDtypeStruct((_N, _D), jnp.float32),
)

_tc_layer2 = pl.pallas_call(
    _tc_layer2_body,
    grid=(_NP // _BR,),
    in_specs=[_row_spec(), _agg_spec(), _deg_spec(), _w_spec(), _w_spec(),
              _b_spec(), _w_spec(), _b_spec()],
    out_specs=_row_spec(),
    out_shape=jax.ShapeDtypeStruct((_N, _D), jnp.float32),
)


def _halves(x):
    return jnp.stack([x[:, :_DH], x[:, _DH:]])


@jax.jit
def kernel(features, edge_index, W_self1, W_neigh1, b1, W_self2, W_neigh2,
           b2, W_fc, b_fc):
    src = edge_index[0].reshape(_NS, _NG, _GS)
    dst = edge_index[1].reshape(_NS, _NG, _GS)
    z_agg = jnp.zeros((_NP, _DH), jnp.float32)

    agg1p, degp = _sc_pass_deg(_halves(features), src, dst, z_agg)
    h1 = _tc_layer1(features, agg1p, degp, W_self1, W_neigh1,
                    b1.reshape(1, _D))
    agg2p = _sc_pass(_halves(h1), src, dst, z_agg)
    out = _tc_layer2(h1, agg2p, degp, W_self2, W_neigh2, b2.reshape(1, _D),
                     W_fc, b_fc.reshape(1, _D))
    return out
